# baseline (speedup 1.0000x reference)
import jax
import jax.numpy as jnp
from jax.experimental import pallas as pl
from jax.experimental.pallas import tpu as pltpu

B = 256
C = 2048
P = 128
NUM_CLASSES = 10000
SCALE = 1.0

_POOL_ROWS = 16384
_GROUP = 128

_NT = 1024
_NC = (NUM_CLASSES + _NT - 1) // _NT


def _pool_kernel(x_ref, o_ref):
    inv = jnp.float32(1.0 / P)
    for g in range(_POOL_ROWS // _GROUP):
        x2 = x_ref[g * _GROUP:(g + 1) * _GROUP, :]
        s = jnp.sum(x2, axis=1) * inv
        o_ref[g:g + 1, :] = s[None, :]


def _head_kernel(t_ref, neck_ref, w_ref, o1_ref, o2_ref, mx_ref, se_ref,
                 lt_ref):
    n = pl.program_id(0)
    tile = jax.lax.dot_general(
        neck_ref[...], w_ref[...],
        dimension_numbers=(((1,), (1,)), ((), ())),
        preferred_element_type=jnp.float32)
    o1_ref[...] = tile
    o2_ref[...] = tile

    col0 = n * _NT
    lane = jax.lax.broadcasted_iota(jnp.int32, (B, _NT), 1)
    neg = jnp.float32(-jnp.inf)
    masked = jnp.where(col0 + lane < NUM_CLASSES, tile, neg)
    cmax = jnp.max(masked, axis=1, keepdims=True)
    sexp = jnp.sum(jnp.exp(masked - cmax), axis=1, keepdims=True)
    hit = lane == (t_ref[...] - col0)
    ltv = jnp.max(jnp.where(hit, tile, neg), axis=1, keepdims=True)
    mx_ref[...] = cmax[None]
    se_ref[...] = sexp[None]
    lt_ref[...] = ltv[None]


def _final_kernel(mx_ref, se_ref, lt_ref, nw_ref):
    m = mx_ref[...]
    gmax = jnp.max(m, axis=0)
    den = jnp.sum(se_ref[...] * jnp.exp(m - gmax[None]), axis=0)
    lt = jnp.max(lt_ref[...], axis=0)
    right_prob = jnp.exp(lt - gmax) / den
    var_sl = jnp.zeros_like(right_prob)
    con = right_prob / (var_sl * 1e4)
    ri = jnp.tanh(1.2 * con)
    nw_ref[...] = (jnp.float32(B) * ri) / jnp.sum(ri, axis=0, keepdims=True)


def kernel(features, targets, weight):
    x = features.reshape(B * C, P)
    pooled = pl.pallas_call(
        _pool_kernel,
        grid=(B * C // _POOL_ROWS,),
        in_specs=[pl.BlockSpec((_POOL_ROWS, P), lambda i: (i, 0))],
        out_specs=pl.BlockSpec((_POOL_ROWS // _GROUP, P), lambda i: (i, 0)),
        out_shape=jax.ShapeDtypeStruct((B * C // _GROUP, P), jnp.float32),
        compiler_params=pltpu.CompilerParams(
            dimension_semantics=("parallel",)),
        name="mean_pool",
    )(x)
    neck = pooled.reshape(B, C)

    t2 = targets.astype(jnp.int32).reshape(B, 1)
    o1, o2, mx, se, lt = pl.pallas_call(
        _head_kernel,
        grid=(_NC,),
        in_specs=[
            pl.BlockSpec((B, 1), lambda n: (0, 0)),
            pl.BlockSpec((B, C), lambda n: (0, 0)),
            pl.BlockSpec((_NT, C), lambda n: (n, 0)),
        ],
        out_specs=[
            pl.BlockSpec((B, _NT), lambda n: (0, n)),
            pl.BlockSpec((B, _NT), lambda n: (0, n)),
            pl.BlockSpec((1, B, 1), lambda n: (n, 0, 0)),
            pl.BlockSpec((1, B, 1), lambda n: (n, 0, 0)),
            pl.BlockSpec((1, B, 1), lambda n: (n, 0, 0)),
        ],
        out_shape=[
            jax.ShapeDtypeStruct((B, NUM_CLASSES), jnp.float32),
            jax.ShapeDtypeStruct((B, NUM_CLASSES), jnp.float32),
            jax.ShapeDtypeStruct((_NC, B, 1), jnp.float32),
            jax.ShapeDtypeStruct((_NC, B, 1), jnp.float32),
            jax.ShapeDtypeStruct((_NC, B, 1), jnp.float32),
        ],
        compiler_params=pltpu.CompilerParams(
            dimension_semantics=("parallel",),
            vmem_limit_bytes=48 * 1024 * 1024),
        name="linear_softmax_stats",
    )(t2, neck, weight)

    nw = pl.pallas_call(
        _final_kernel,
        out_shape=jax.ShapeDtypeStruct((B, 1), jnp.float32),
        name="reweight_finalize",
    )(mx, se, lt)

    return o1, o2, neck, nw.reshape(1, B)


# speedup vs baseline: 5.9415x; 5.9415x over previous
import jax
import jax.numpy as jnp
from jax.experimental import pallas as pl
from jax.experimental.pallas import tpu as pltpu

B = 256
C = 2048
P = 128
NUM_CLASSES = 10000
SCALE = 1.0

_BB = 8

_NT = 1024
_NC = (NUM_CLASSES + _NT - 1) // _NT


def _pool_kernel(x_ref, o_ref):
    o_ref[...] = jnp.mean(x_ref[...], axis=1)


def _head_kernel(t_ref, neck_ref, w_ref, o1_ref, o2_ref, mx_ref, se_ref,
                 lt_ref):
    n = pl.program_id(0)
    tile = jax.lax.dot_general(
        w_ref[...], neck_ref[...],
        dimension_numbers=(((1,), (1,)), ((), ())),
        preferred_element_type=jnp.float32)
    o1_ref[...] = tile
    o2_ref[...] = tile

    row0 = n * _NT
    row = jax.lax.broadcasted_iota(jnp.int32, (_NT, B), 0)
    neg = jnp.float32(-jnp.inf)
    masked = jnp.where(row0 + row < NUM_CLASSES, tile, neg)
    cmax = jnp.max(masked, axis=0, keepdims=True)
    sexp = jnp.sum(jnp.exp(masked - cmax), axis=0, keepdims=True)
    hit = row == (t_ref[...] - row0)
    ltv = jnp.max(jnp.where(hit, tile, neg), axis=0, keepdims=True)
    mx_ref[...] = cmax[None]
    se_ref[...] = sexp[None]
    lt_ref[...] = ltv[None]


def _final_kernel(mx_ref, se_ref, lt_ref, nw_ref):
    m = mx_ref[...]
    gmax = jnp.max(m, axis=0)
    den = jnp.sum(se_ref[...] * jnp.exp(m - gmax[None]), axis=0)
    lt = jnp.max(lt_ref[...], axis=0)
    right_prob = jnp.exp(lt - gmax) / den
    var_sl = jnp.zeros_like(right_prob)
    con = right_prob / (var_sl * 1e4)
    ri = jnp.tanh(1.2 * con)
    nw_ref[...] = (jnp.float32(B) * ri) / jnp.sum(ri, axis=1, keepdims=True)


def kernel(features, targets, weight):
    x = features.transpose(0, 2, 3, 1).reshape(B, P, C)
    neck = pl.pallas_call(
        _pool_kernel,
        grid=(B // _BB,),
        in_specs=[pl.BlockSpec((_BB, P, C), lambda i: (i, 0, 0))],
        out_specs=pl.BlockSpec((_BB, C), lambda i: (i, 0)),
        out_shape=jax.ShapeDtypeStruct((B, C), jnp.float32),
        compiler_params=pltpu.CompilerParams(
            dimension_semantics=("parallel",)),
        name="mean_pool",
    )(x)

    t2 = targets.astype(jnp.int32).reshape(1, B)
    o1, o2, mx, se, lt = pl.pallas_call(
        _head_kernel,
        grid=(_NC,),
        in_specs=[
            pl.BlockSpec((1, B), lambda n: (0, 0)),
            pl.BlockSpec((B, C), lambda n: (0, 0)),
            pl.BlockSpec((_NT, C), lambda n: (n, 0)),
        ],
        out_specs=[
            pl.BlockSpec((_NT, B), lambda n: (n, 0)),
            pl.BlockSpec((_NT, B), lambda n: (n, 0)),
            pl.BlockSpec((1, 1, B), lambda n: (n, 0, 0)),
            pl.BlockSpec((1, 1, B), lambda n: (n, 0, 0)),
            pl.BlockSpec((1, 1, B), lambda n: (n, 0, 0)),
        ],
        out_shape=[
            jax.ShapeDtypeStruct((NUM_CLASSES, B), jnp.float32),
            jax.ShapeDtypeStruct((NUM_CLASSES, B), jnp.float32),
            jax.ShapeDtypeStruct((_NC, 1, B), jnp.float32),
            jax.ShapeDtypeStruct((_NC, 1, B), jnp.float32),
            jax.ShapeDtypeStruct((_NC, 1, B), jnp.float32),
        ],
        compiler_params=pltpu.CompilerParams(
            dimension_semantics=("parallel",),
            vmem_limit_bytes=48 * 1024 * 1024),
        name="linear_softmax_stats",
    )(t2, neck, weight)

    nw = pl.pallas_call(
        _final_kernel,
        out_shape=jax.ShapeDtypeStruct((1, B), jnp.float32),
        name="reweight_finalize",
    )(mx, se, lt)

    return o1.T, o2.T, neck, nw
